# Initial kernel scaffold; baseline (speedup 1.0000x reference)
import jax
import jax.numpy as jnp
from jax.experimental import pallas as pl


def kernel(x, tables):
    raise NotImplementedError("write your pallas kernel here")


# speedup vs baseline: 9.8305x; 9.8305x over previous
import functools

import numpy as np
import jax
import jax.numpy as jnp
from jax import lax
from jax.experimental import pallas as pl
from jax.experimental.pallas import tpu as pltpu
from jax.experimental.pallas import tpu_sc as plsc

NUM_FIELDS = 26
FIELD_DIM = 4096
TOTAL_ROWS = NUM_FIELDS * FIELD_DIM
EMBED_DIM = 16
NUM_PAIRS = NUM_FIELDS * (NUM_FIELDS - 1) // 2

NC = 2
NS = 16
NW = NC * NS

CHUNK = 128

_I, _J = np.triu_indices(NUM_FIELDS, k=1)


def _body(t2, ia, ib, out, ia_v, ib_v, ra, rb, ov, sem, *, cpw):
    wid = lax.axis_index("s") * NC + lax.axis_index("c")
    pltpu.sync_copy(ia.at[wid], ia_v)
    pltpu.sync_copy(ib.at[wid], ib_v)
    base = wid * (cpw * CHUNK)

    @pl.loop(0, cpw)
    def _chunk(c):
        ca = pltpu.async_copy(t2.at[ia_v.at[c]], ra, sem)
        cb = pltpu.async_copy(t2.at[ib_v.at[c]], rb, sem)
        ca.wait()
        cb.wait()

        @plsc.parallel_loop(0, CHUNK, unroll=8)
        def _row(r):
            ov[r, :] = ra[r, :] * rb[r, :]

        pltpu.sync_copy(ov, out.at[pl.ds(base + c * CHUNK, CHUNK)])


def kernel(x, tables):
    batch = x.shape[0]
    xi = x.astype(jnp.int32)
    i_idx = jnp.asarray(_I * FIELD_DIM, jnp.int32)
    j_idx = jnp.asarray(_J * FIELD_DIM, jnp.int32)
    base_a = jnp.asarray(_J * TOTAL_ROWS, jnp.int32) + i_idx
    base_b = jnp.asarray(_I * TOTAL_ROWS, jnp.int32) + j_idx
    idx_a = base_a[None, :] + xi[:, _I]
    idx_b = base_b[None, :] + xi[:, _J]

    rows = batch * NUM_PAIRS
    cpw = rows // (NW * CHUNK)
    assert cpw * NW * CHUNK == rows
    ia = idx_a.reshape(NW, cpw, CHUNK)
    ib = idx_b.reshape(NW, cpw, CHUNK)
    t2 = tables.reshape(NUM_FIELDS * TOTAL_ROWS, EMBED_DIM)

    fn = pl.kernel(
        functools.partial(_body, cpw=cpw),
        out_type=jax.ShapeDtypeStruct((rows, EMBED_DIM), jnp.float32),
        mesh=plsc.VectorSubcoreMesh(core_axis_name="c", subcore_axis_name="s"),
        scratch_types=[
            pltpu.VMEM((cpw, CHUNK), jnp.int32),
            pltpu.VMEM((cpw, CHUNK), jnp.int32),
            pltpu.VMEM((CHUNK, EMBED_DIM), jnp.float32),
            pltpu.VMEM((CHUNK, EMBED_DIM), jnp.float32),
            pltpu.VMEM((CHUNK, EMBED_DIM), jnp.float32),
            pltpu.SemaphoreType.DMA,
        ],
        compiler_params=pltpu.CompilerParams(use_tc_tiling_on_sc=False),
    )
    out = fn(t2, ia, ib)
    return out.reshape(batch, NUM_PAIRS, EMBED_DIM)


# speedup vs baseline: 10.6499x; 1.0833x over previous
import functools

import numpy as np
import jax
import jax.numpy as jnp
from jax import lax
from jax.experimental import pallas as pl
from jax.experimental.pallas import tpu as pltpu
from jax.experimental.pallas import tpu_sc as plsc

NUM_FIELDS = 26
FIELD_DIM = 4096
TOTAL_ROWS = NUM_FIELDS * FIELD_DIM
EMBED_DIM = 16
NUM_PAIRS = NUM_FIELDS * (NUM_FIELDS - 1) // 2

NC = 2
NS = 16
NW = NC * NS

CHUNK = 128

_I, _J = np.triu_indices(NUM_FIELDS, k=1)


NBUF = 5


def _body(t2, ia, ib, out, ia_v, ib_v, ra, rb, ov, semg, semo, *, cpw):
    wid = lax.axis_index("s") * NC + lax.axis_index("c")
    pltpu.sync_copy(ia.at[wid], ia_v)
    pltpu.sync_copy(ib.at[wid], ib_v)
    base = wid * (cpw * CHUNK)

    def fire(c, b):
        pltpu.async_copy(t2.at[ia_v.at[c]], ra.at[b], semg.at[b])
        pltpu.async_copy(t2.at[ib_v.at[c]], rb.at[b], semg.at[b])

    for b in range(NBUF - 1):
        fire(b, b)

    @pl.loop(0, cpw, step=NBUF)
    def _outer(o):
        for b in range(NBUF):
            c = o + b
            pltpu.make_async_copy(t2.at[ia_v.at[c]], ra.at[b], semg.at[b]).wait()
            pltpu.make_async_copy(t2.at[ib_v.at[c]], rb.at[b], semg.at[b]).wait()

            @pl.when(c + NBUF - 1 < cpw)
            def _():
                fire(c + NBUF - 1, (b + NBUF - 1) % NBUF)

            @pl.when(o > 0)
            def _():
                pltpu.make_async_copy(
                    ov.at[b], out.at[pl.ds(base, CHUNK)], semo.at[b]
                ).wait()

            @plsc.parallel_loop(0, CHUNK, unroll=8)
            def _row(r):
                ov[b, r, :] = ra[b, r, :] * rb[b, r, :]

            pltpu.async_copy(
                ov.at[b], out.at[pl.ds(base + c * CHUNK, CHUNK)], semo.at[b]
            )

    for b in range(NBUF):
        pltpu.make_async_copy(
            ov.at[b], out.at[pl.ds(base, CHUNK)], semo.at[b]
        ).wait()


def kernel(x, tables):
    batch = x.shape[0]
    xi = x.astype(jnp.int32)
    i_idx = jnp.asarray(_I * FIELD_DIM, jnp.int32)
    j_idx = jnp.asarray(_J * FIELD_DIM, jnp.int32)
    base_a = jnp.asarray(_J * TOTAL_ROWS, jnp.int32) + i_idx
    base_b = jnp.asarray(_I * TOTAL_ROWS, jnp.int32) + j_idx
    idx_a = base_a[None, :] + xi[:, _I]
    idx_b = base_b[None, :] + xi[:, _J]

    rows = batch * NUM_PAIRS
    cpw = rows // (NW * CHUNK)
    assert cpw * NW * CHUNK == rows
    ia = idx_a.reshape(NW, cpw, CHUNK)
    ib = idx_b.reshape(NW, cpw, CHUNK)
    t2 = tables.reshape(NUM_FIELDS * TOTAL_ROWS, EMBED_DIM)

    fn = pl.kernel(
        functools.partial(_body, cpw=cpw),
        out_type=jax.ShapeDtypeStruct((rows, EMBED_DIM), jnp.float32),
        mesh=plsc.VectorSubcoreMesh(core_axis_name="c", subcore_axis_name="s"),
        scratch_types=[
            pltpu.VMEM((cpw, CHUNK), jnp.int32),
            pltpu.VMEM((cpw, CHUNK), jnp.int32),
            pltpu.VMEM((NBUF, CHUNK, EMBED_DIM), jnp.float32),
            pltpu.VMEM((NBUF, CHUNK, EMBED_DIM), jnp.float32),
            pltpu.VMEM((NBUF, CHUNK, EMBED_DIM), jnp.float32),
            pltpu.SemaphoreType.DMA((NBUF,)),
            pltpu.SemaphoreType.DMA((NBUF,)),
        ],
        compiler_params=pltpu.CompilerParams(use_tc_tiling_on_sc=False),
    )
    out = fn(t2, ia, ib)
    return out.reshape(batch, NUM_PAIRS, EMBED_DIM)


# speedup vs baseline: 91.3542x; 8.5780x over previous
import functools

import numpy as np
import jax
import jax.numpy as jnp
from jax import lax
from jax.experimental import pallas as pl
from jax.experimental.pallas import tpu as pltpu
from jax.experimental.pallas import tpu_sc as plsc

NUM_FIELDS = 26
FIELD_DIM = 4096
TOTAL_ROWS = NUM_FIELDS * FIELD_DIM
EMBED_DIM = 16
NUM_PAIRS = NUM_FIELDS * (NUM_FIELDS - 1) // 2
NUM_BLOCKS = NUM_PAIRS * 2

NC = 2
NS = 16
NW = NC * NS

RT = FIELD_DIM // 128
KMAX = -(-NUM_BLOCKS // NW)

def _pair_decode(p):
    def step(_, carry):
        i_cur, off = carry
        nxt = off + (NUM_FIELDS - 1 - i_cur)
        take = nxt <= p
        return (jnp.where(take, i_cur + 1, i_cur), jnp.where(take, nxt, off))

    i, off = lax.fori_loop(0, NUM_FIELDS - 1, step, (jnp.int32(0), jnp.int32(0)))
    j = i + 1 + (p - off)
    return i, j


def _body(tv, xt, out, xa_v, xb_v, a_v, b_v, ov, sem_in, sem_out):
    wid = lax.axis_index("s") * NC + lax.axis_index("c")

    @pl.loop(0, KMAX)
    def _k(k):
        g = k * NW + wid
        dt = lax.rem(g, 2)
        p = lax.div(g, 2)
        i, j = _pair_decode(p)
        row_a = (j * 2 + dt) * (RT * NUM_FIELDS) + i * RT
        row_b = (i * 2 + dt) * (RT * NUM_FIELDS) + j * RT

        @pl.when(g < NUM_BLOCKS)
        def _block():
            ca = pltpu.async_copy(tv.at[pl.ds(row_a * 1024, RT * 1024)],
                                  a_v, sem_in)
            cb = pltpu.async_copy(tv.at[pl.ds(row_b * 1024, RT * 1024)],
                                  b_v, sem_in)
            cx = pltpu.async_copy(xt.at[i], xa_v, sem_in)
            cy = pltpu.async_copy(xt.at[j], xb_v, sem_in)
            ca.wait()
            cb.wait()
            cx.wait()
            cy.wait()

            @pl.when(k > 0)
            def _():
                pltpu.make_async_copy(ov, out.at[0], sem_out).wait()

            @pl.loop(0, RT)
            def _bt(bt):
                for s in range(8):
                    xi = xa_v[pl.ds(bt * 128 + s * 16, 16)]
                    xj = xb_v[pl.ds(bt * 128 + s * 16, 16)]
                    fa = jax.lax.shift_left(
                        jax.lax.shift_right_logical(xi, 7), 10) \
                        + jax.lax.bitwise_and(xi, 127)
                    fb = jax.lax.shift_left(
                        jax.lax.shift_right_logical(xj, 7), 10) \
                        + jax.lax.bitwise_and(xj, 127)
                    for dl in range(8):
                        va = plsc.load_gather(a_v, [fa + dl * 128])
                        vb = plsc.load_gather(b_v, [fb + dl * 128])
                        ov[pl.ds(bt * 1024 + dl * 128 + s * 16, 16)] = va * vb

            pltpu.async_copy(ov, out.at[g], sem_out)

    pltpu.make_async_copy(ov, out.at[0], sem_out).wait()


def kernel(x, tables):
    batch = x.shape[0]
    xt = x.astype(jnp.int32).T
    tv = (tables.reshape(NUM_FIELDS, TOTAL_ROWS // 128, 128, 2, 8)
          .transpose(0, 3, 1, 4, 2)
          .reshape(NUM_FIELDS * 2 * (TOTAL_ROWS // 128) * 1024))
    fn = pl.kernel(
        _body,
        out_type=jax.ShapeDtypeStruct((NUM_BLOCKS, batch // 128 * 1024),
                                      jnp.float32),
        mesh=plsc.VectorSubcoreMesh(core_axis_name="c", subcore_axis_name="s"),
        scratch_types=[
            pltpu.VMEM((batch,), jnp.int32),
            pltpu.VMEM((batch,), jnp.int32),
            pltpu.VMEM((RT * 1024,), jnp.float32),
            pltpu.VMEM((RT * 1024,), jnp.float32),
            pltpu.VMEM((batch // 128 * 1024,), jnp.float32),
            pltpu.SemaphoreType.DMA,
            pltpu.SemaphoreType.DMA,
        ],
        compiler_params=pltpu.CompilerParams(use_tc_tiling_on_sc=False,
                                             needs_layout_passes=False),
    )
    o4 = fn(tv, xt)
    out = (o4.reshape(NUM_PAIRS, 2, batch // 128, 8, 128)
           .transpose(2, 4, 0, 1, 3)
           .reshape(batch, NUM_PAIRS, EMBED_DIM))
    return out


# speedup vs baseline: 184.1361x; 2.0156x over previous
import functools

import numpy as np
import jax
import jax.numpy as jnp
from jax import lax
from jax.experimental import pallas as pl
from jax.experimental.pallas import tpu as pltpu
from jax.experimental.pallas import tpu_sc as plsc

NUM_FIELDS = 26
FIELD_DIM = 4096
TOTAL_ROWS = NUM_FIELDS * FIELD_DIM
EMBED_DIM = 16
NUM_PAIRS = NUM_FIELDS * (NUM_FIELDS - 1) // 2
NUM_BLOCKS = NUM_PAIRS * 2

NC = 2
NS = 16
NW = NC * NS

RT = FIELD_DIM // 128
KMAX = -(-NUM_BLOCKS // NW)

def _pair_decode(p):
    def step(_, carry):
        i_cur, off = carry
        nxt = off + (NUM_FIELDS - 1 - i_cur)
        take = nxt <= p
        return (jnp.where(take, i_cur + 1, i_cur), jnp.where(take, nxt, off))

    i, off = lax.fori_loop(0, NUM_FIELDS - 1, step, (jnp.int32(0), jnp.int32(0)))
    j = i + 1 + (p - off)
    return i, j


def _body(tv, xt, out, xa_v, xb_v, a_v, b_v, ov, sem_in, sem_out):
    wid = lax.axis_index("s") * NC + lax.axis_index("c")

    @pl.loop(0, KMAX)
    def _k(k):
        g = k * NW + wid
        dt = lax.rem(g, 2)
        p = lax.div(g, 2)
        i, j = _pair_decode(p)
        row_a = (j * 2 + dt) * (RT * NUM_FIELDS) + i * RT
        row_b = (i * 2 + dt) * (RT * NUM_FIELDS) + j * RT

        @pl.when(g < NUM_BLOCKS)
        def _block():
            ca = pltpu.async_copy(tv.at[pl.ds(row_a * 1024, RT * 1024)],
                                  a_v, sem_in)
            cb = pltpu.async_copy(tv.at[pl.ds(row_b * 1024, RT * 1024)],
                                  b_v, sem_in)
            cx = pltpu.async_copy(xt.at[i], xa_v, sem_in)
            cy = pltpu.async_copy(xt.at[j], xb_v, sem_in)
            ca.wait()
            cb.wait()
            cx.wait()
            cy.wait()

            @pl.when(k > 0)
            def _():
                pltpu.make_async_copy(ov, out.at[0], sem_out).wait()

            @plsc.parallel_loop(0, RT, unroll=2)
            def _bt(bt):
                for s in range(8):
                    xi = xa_v[pl.ds(bt * 128 + s * 16, 16)]
                    xj = xb_v[pl.ds(bt * 128 + s * 16, 16)]
                    fa = jax.lax.shift_left(
                        jax.lax.shift_right_logical(xi, 7), 10) \
                        + jax.lax.bitwise_and(xi, 127)
                    fb = jax.lax.shift_left(
                        jax.lax.shift_right_logical(xj, 7), 10) \
                        + jax.lax.bitwise_and(xj, 127)
                    for dl in range(8):
                        va = plsc.load_gather(a_v, [fa + dl * 128])
                        vb = plsc.load_gather(b_v, [fb + dl * 128])
                        ov[pl.ds(bt * 1024 + dl * 128 + s * 16, 16)] = va * vb

            pltpu.async_copy(ov, out.at[g], sem_out)

    pltpu.make_async_copy(ov, out.at[0], sem_out).wait()


def kernel(x, tables):
    batch = x.shape[0]
    xt = x.astype(jnp.int32).T
    tv = (tables.reshape(NUM_FIELDS, TOTAL_ROWS // 128, 128, 2, 8)
          .transpose(0, 3, 1, 4, 2)
          .reshape(NUM_FIELDS * 2 * (TOTAL_ROWS // 128) * 1024))
    fn = pl.kernel(
        _body,
        out_type=jax.ShapeDtypeStruct((NUM_BLOCKS, batch // 128 * 1024),
                                      jnp.float32),
        mesh=plsc.VectorSubcoreMesh(core_axis_name="c", subcore_axis_name="s"),
        scratch_types=[
            pltpu.VMEM((batch,), jnp.int32),
            pltpu.VMEM((batch,), jnp.int32),
            pltpu.VMEM((RT * 1024,), jnp.float32),
            pltpu.VMEM((RT * 1024,), jnp.float32),
            pltpu.VMEM((batch // 128 * 1024,), jnp.float32),
            pltpu.SemaphoreType.DMA,
            pltpu.SemaphoreType.DMA,
        ],
        compiler_params=pltpu.CompilerParams(use_tc_tiling_on_sc=False,
                                             needs_layout_passes=False),
    )
    o4 = fn(tv, xt)
    out = (o4.reshape(NUM_PAIRS, 2, batch // 128, 8, 128)
           .transpose(2, 4, 0, 1, 3)
           .reshape(batch, NUM_PAIRS, EMBED_DIM))
    return out
